# baseline (device time: 18529 ns/iter reference)
import jax
import jax.numpy as jnp
from jax import lax
from jax.experimental import pallas as pl
from jax.experimental.pallas import tpu as pltpu

N_DEV = 16
CH = 2


def kernel(x):
    _, m, n = x.shape
    rows = m // N_DEV
    rc = rows // CH

    def body(x_ref, out_ref, rs_buf, rs_send, rs_recv, ag_send, ag_recv):
        my = lax.axis_index("i")

        barrier_sem = pltpu.get_barrier_semaphore()
        for r in range(1, N_DEV):
            pl.semaphore_signal(
                barrier_sem, inc=1,
                device_id=(lax.rem(my + r, N_DEV),),
                device_id_type=pl.DeviceIdType.MESH,
            )
        pl.semaphore_wait(barrier_sem, N_DEV - 1)

        for c in range(CH):
            for r in range(1, N_DEV):
                tgt = lax.rem(my + r, N_DEV)
                pltpu.make_async_remote_copy(
                    src_ref=x_ref.at[pl.ds(tgt * rows + c * rc, rc), :],
                    dst_ref=rs_buf.at[N_DEV - r, pl.ds(c * rc, rc), :],
                    send_sem=rs_send.at[c, r],
                    recv_sem=rs_recv.at[c, N_DEV - r],
                    device_id=(tgt,),
                    device_id_type=pl.DeviceIdType.MESH,
                ).start()

        for c in range(CH):
            for s in range(1, N_DEV):
                pltpu.make_async_copy(
                    rs_buf.at[s, pl.ds(c * rc, rc), :],
                    rs_buf.at[s, pl.ds(c * rc, rc), :],
                    rs_recv.at[c, s],
                ).wait()
            own = x_ref[pl.ds(my * rows + c * rc, rc), :]
            peer_sum = jnp.sum(rs_buf[1:, c * rc:(c + 1) * rc, :], axis=0)
            out_ref[pl.ds(my * rows + c * rc, rc), :] = own + peer_sum

            for r in range(1, N_DEV):
                tgt = lax.rem(my + r, N_DEV)
                pltpu.make_async_remote_copy(
                    src_ref=out_ref.at[pl.ds(my * rows + c * rc, rc), :],
                    dst_ref=out_ref.at[pl.ds(my * rows + c * rc, rc), :],
                    send_sem=ag_send.at[c, r],
                    recv_sem=ag_recv.at[c, N_DEV - r],
                    device_id=(tgt,),
                    device_id_type=pl.DeviceIdType.MESH,
                ).start()

        for c in range(CH):
            for s in range(1, N_DEV):
                pltpu.make_async_copy(
                    out_ref.at[pl.ds(0, rc), :],
                    out_ref.at[pl.ds(0, rc), :],
                    ag_recv.at[c, s],
                ).wait()
            for r in range(1, N_DEV):
                pltpu.make_async_copy(
                    out_ref.at[pl.ds(0, rc), :],
                    out_ref.at[pl.ds(0, rc), :],
                    rs_send.at[c, r],
                ).wait()
                pltpu.make_async_copy(
                    out_ref.at[pl.ds(0, rc), :],
                    out_ref.at[pl.ds(0, rc), :],
                    ag_send.at[c, r],
                ).wait()

    x2 = x.reshape(m, n)
    return pl.pallas_call(
        body,
        out_shape=jax.ShapeDtypeStruct((m, n), x.dtype),
        in_specs=[pl.BlockSpec(memory_space=pltpu.VMEM)],
        out_specs=pl.BlockSpec(memory_space=pltpu.VMEM),
        scratch_shapes=[
            pltpu.VMEM((N_DEV, rows, n), x.dtype),
            pltpu.SemaphoreType.DMA((CH, N_DEV)),
            pltpu.SemaphoreType.DMA((CH, N_DEV)),
            pltpu.SemaphoreType.DMA((CH, N_DEV)),
            pltpu.SemaphoreType.DMA((CH, N_DEV)),
        ],
        compiler_params=pltpu.CompilerParams(collective_id=0),
    )(x2)


# device time: 15031 ns/iter; 1.2327x vs baseline; 1.2327x over previous
import jax
import jax.numpy as jnp
from jax import lax
from jax.experimental import pallas as pl
from jax.experimental.pallas import tpu as pltpu

N_DEV = 16
CH = 2

_FAR_FIRST = sorted(range(1, N_DEV), key=lambda r: -min(r, N_DEV - r))
_NEAR_FIRST_SLOTS = sorted(range(1, N_DEV), key=lambda s: min(s, N_DEV - s))


def kernel(x):
    _, m, n = x.shape
    rows = m // N_DEV
    rc = rows // CH

    def body(x_ref, out_ref, rs_buf, rs_send, rs_recv, ag_send, ag_recv):
        my = lax.axis_index("i")

        barrier_sem = pltpu.get_barrier_semaphore()
        for r in range(1, N_DEV):
            pl.semaphore_signal(
                barrier_sem, inc=1,
                device_id=(lax.rem(my + r, N_DEV),),
                device_id_type=pl.DeviceIdType.MESH,
            )
        pl.semaphore_wait(barrier_sem, N_DEV - 1)

        for c in range(CH):
            for r in _FAR_FIRST:
                tgt = lax.rem(my + r, N_DEV)
                pltpu.make_async_remote_copy(
                    src_ref=x_ref.at[0, pl.ds(tgt * rows + c * rc, rc), :],
                    dst_ref=rs_buf.at[N_DEV - r, pl.ds(c * rc, rc), :],
                    send_sem=rs_send.at[c, r],
                    recv_sem=rs_recv.at[c, N_DEV - r],
                    device_id=(tgt,),
                    device_id_type=pl.DeviceIdType.MESH,
                ).start()

        for c in range(CH):
            acc = x_ref[0, pl.ds(my * rows + c * rc, rc), :]
            for s in _NEAR_FIRST_SLOTS:
                pltpu.make_async_copy(
                    rs_buf.at[s, pl.ds(c * rc, rc), :],
                    rs_buf.at[s, pl.ds(c * rc, rc), :],
                    rs_recv.at[c, s],
                ).wait()
                acc = acc + rs_buf[s, pl.ds(c * rc, rc), :]
            out_ref[pl.ds(my * rows + c * rc, rc), :] = acc

            for r in _FAR_FIRST:
                tgt = lax.rem(my + r, N_DEV)
                pltpu.make_async_remote_copy(
                    src_ref=out_ref.at[pl.ds(my * rows + c * rc, rc), :],
                    dst_ref=out_ref.at[pl.ds(my * rows + c * rc, rc), :],
                    send_sem=ag_send.at[c, r],
                    recv_sem=ag_recv.at[c, N_DEV - r],
                    device_id=(tgt,),
                    device_id_type=pl.DeviceIdType.MESH,
                ).start()

        for c in range(CH):
            for s in _NEAR_FIRST_SLOTS:
                pltpu.make_async_copy(
                    out_ref.at[pl.ds(0, rc), :],
                    out_ref.at[pl.ds(0, rc), :],
                    ag_recv.at[c, s],
                ).wait()
            for r in range(1, N_DEV):
                pltpu.make_async_copy(
                    out_ref.at[pl.ds(0, rc), :],
                    out_ref.at[pl.ds(0, rc), :],
                    rs_send.at[c, r],
                ).wait()
                pltpu.make_async_copy(
                    out_ref.at[pl.ds(0, rc), :],
                    out_ref.at[pl.ds(0, rc), :],
                    ag_send.at[c, r],
                ).wait()

    return pl.pallas_call(
        body,
        out_shape=jax.ShapeDtypeStruct((m, n), x.dtype),
        in_specs=[pl.BlockSpec(memory_space=pltpu.VMEM)],
        out_specs=pl.BlockSpec(memory_space=pltpu.VMEM),
        scratch_shapes=[
            pltpu.VMEM((N_DEV, rows, n), x.dtype),
            pltpu.SemaphoreType.DMA((CH, N_DEV)),
            pltpu.SemaphoreType.DMA((CH, N_DEV)),
            pltpu.SemaphoreType.DMA((CH, N_DEV)),
            pltpu.SemaphoreType.DMA((CH, N_DEV)),
        ],
        compiler_params=pltpu.CompilerParams(collective_id=0),
    )(x)
